# baseline (speedup 1.0000x reference)
import functools

import jax
import jax.numpy as jnp
from jax import lax
from jax.experimental import pallas as pl
from jax.experimental.pallas import tpu as pltpu
from jax.experimental.pallas import tpu_sc as plsc

B, R, F = 1024, 200, 128
G, L = 4, 32
N = B * R
OUT = G * L
LANES = 16

NUM_CORES = 2
NUM_SUBCORES = 16
NW = NUM_CORES * NUM_SUBCORES
ROWS_PER_W = N // NW
CHUNK_ROWS = 128
NUM_CHUNKS = ROWS_PER_W // CHUNK_ROWS


def _sc_body(x_hbm, gi_hbm, gm_hbm, out_hbm,
             in_v0, in_v1, out_v0, out_v1, idx_v, msk_v,
             si0, si1, so0, so1):
    in_bufs = (in_v0, in_v1)
    out_bufs = (out_v0, out_v1)
    sin = (si0, si1)
    sout = (so0, so1)

    wid = lax.axis_index("s") * NUM_CORES + lax.axis_index("c")
    row0_w = wid * ROWS_PER_W

    pltpu.sync_copy(gi_hbm, idx_v)
    pltpu.sync_copy(gm_hbm, msk_v)

    cols = []
    msks = []
    for j in range(OUT // LANES):
        cj = idx_v[pl.ds(j * LANES, LANES)]
        cj = jnp.minimum(jnp.maximum(cj, 0), F - 1)
        cols.append(cj)
        msks.append(msk_v[pl.ds(j * LANES, LANES)])

    def in_dma(ci, b):
        row0 = row0_w + ci * CHUNK_ROWS
        return pltpu.make_async_copy(
            x_hbm.at[pl.ds(row0 * F, CHUNK_ROWS * F)], in_bufs[b], sin[b])

    def out_dma(ci, b):
        row0 = row0_w + ci * CHUNK_ROWS
        return pltpu.make_async_copy(
            out_bufs[b], out_hbm.at[pl.ds(row0 * OUT, CHUNK_ROWS * OUT)],
            sout[b])

    in_dma(0, 0).start()
    in_dma(1, 1).start()

    def outer(oi, carry):
        for b in range(2):
            ci = 2 * oi + b
            in_dma(ci, b).wait()

            @pl.when(oi > 0)
            def _():
                out_dma(ci - 2, b).wait()

            @plsc.parallel_loop(0, CHUNK_ROWS * F, step=F, unroll=8)
            def row_body(o):
                roff = pl.multiple_of(o, F)
                base = jnp.full((LANES,), roff, dtype=jnp.int32)
                for j in range(OUT // LANES):
                    v = plsc.load_gather(in_bufs[b], [cols[j] + base])
                    out_bufs[b][pl.ds(roff + j * LANES, LANES)] = v * msks[j]

            out_dma(ci, b).start()

            @pl.when(ci + 2 < NUM_CHUNKS)
            def _():
                in_dma(ci + 2, b).start()
        return carry

    lax.fori_loop(0, NUM_CHUNKS // 2, outer, 0, unroll=False)
    out_dma(NUM_CHUNKS - 2, 0).wait()
    out_dma(NUM_CHUNKS - 1, 1).wait()


@jax.jit
def kernel(x_brd, g_idx, g_mask):
    x_flat = x_brd.reshape(N * F)
    gi = g_idx.reshape(OUT)
    gm = g_mask.reshape(OUT)

    mesh = plsc.VectorSubcoreMesh(
        core_axis_name="c", subcore_axis_name="s",
        num_cores=NUM_CORES, num_subcores=NUM_SUBCORES)
    out = pl.kernel(
        _sc_body,
        out_type=jax.ShapeDtypeStruct((N * OUT,), jnp.float32),
        mesh=mesh,
        compiler_params=pltpu.CompilerParams(needs_layout_passes=False),
        scratch_types=[
            pltpu.VMEM((CHUNK_ROWS * F,), jnp.float32),
            pltpu.VMEM((CHUNK_ROWS * F,), jnp.float32),
            pltpu.VMEM((CHUNK_ROWS * OUT,), jnp.float32),
            pltpu.VMEM((CHUNK_ROWS * OUT,), jnp.float32),
            pltpu.VMEM((OUT,), jnp.int32),
            pltpu.VMEM((OUT,), jnp.float32),
            pltpu.SemaphoreType.DMA,
            pltpu.SemaphoreType.DMA,
            pltpu.SemaphoreType.DMA,
            pltpu.SemaphoreType.DMA,
        ],
    )(x_flat, gi, gm)
    return out.reshape(B, R, G, L)


# speedup vs baseline: 2.8351x; 2.8351x over previous
import jax
import jax.numpy as jnp
from jax import lax
from jax.experimental import pallas as pl
from jax.experimental.pallas import tpu as pltpu
from jax.experimental.pallas import tpu_sc as plsc

B, R, F = 1024, 200, 128
G, L = 4, 32
OUT = G * L
LANES = 16

NUM_CORES = 2
NUM_SUBCORES = 16
NW = NUM_CORES * NUM_SUBCORES
B_PER_W = B // NW


def _sc_body(x_hbm, gi_hbm, gm_hbm, out_hbm,
             in_v0, in_v1, out_v0, out_v1, idx_v, msk_v,
             si0, si1, so0, so1):
    in_bufs = (in_v0, in_v1)
    out_bufs = (out_v0, out_v1)
    sin = (si0, si1)
    sout = (so0, so1)

    wid = lax.axis_index("s") * NUM_CORES + lax.axis_index("c")
    b0_w = wid * B_PER_W

    pltpu.sync_copy(gi_hbm, idx_v)
    pltpu.sync_copy(gm_hbm, msk_v)

    cols = []
    msks = []
    for j in range(OUT // LANES):
        g, h = j // 2, j % 2
        cj = idx_v[g, pl.ds(h * LANES, LANES)]
        cj = jnp.minimum(jnp.maximum(cj, 0), F - 1)
        cols.append(cj)
        msks.append(msk_v[g, pl.ds(h * LANES, LANES)])

    def in_dma(ci, bf):
        return pltpu.make_async_copy(
            x_hbm.at[b0_w + ci], in_bufs[bf], sin[bf])

    def out_dma(ci, bf):
        return pltpu.make_async_copy(
            out_bufs[bf], out_hbm.at[b0_w + ci], sout[bf])

    in_dma(0, 0).start()
    in_dma(1, 1).start()

    def outer(oi, carry):
        for bf in range(2):
            ci = 2 * oi + bf
            in_dma(ci, bf).wait()

            @pl.when(oi > 0)
            def _():
                out_dma(ci - 2, bf).wait()

            @plsc.parallel_loop(0, R, step=1, unroll=8)
            def row_body(r):
                base = jnp.full((LANES,), r, dtype=jnp.int32)
                for j in range(OUT // LANES):
                    v = plsc.load_gather(in_bufs[bf], [base, cols[j]])
                    out_bufs[bf][r, pl.ds(j * LANES, LANES)] = v * msks[j]

            out_dma(ci, bf).start()

            @pl.when(ci + 2 < B_PER_W)
            def _():
                in_dma(ci + 2, bf).start()
        return carry

    lax.fori_loop(0, B_PER_W // 2, outer, 0, unroll=False)
    out_dma(B_PER_W - 2, 0).wait()
    out_dma(B_PER_W - 1, 1).wait()


@jax.jit
def kernel(x_brd, g_idx, g_mask):
    mesh = plsc.VectorSubcoreMesh(
        core_axis_name="c", subcore_axis_name="s",
        num_cores=NUM_CORES, num_subcores=NUM_SUBCORES)
    out = pl.kernel(
        _sc_body,
        out_type=jax.ShapeDtypeStruct((B, R, OUT), jnp.float32),
        mesh=mesh,
        compiler_params=pltpu.CompilerParams(needs_layout_passes=False),
        scratch_types=[
            pltpu.VMEM((R, F), jnp.float32),
            pltpu.VMEM((R, F), jnp.float32),
            pltpu.VMEM((R, OUT), jnp.float32),
            pltpu.VMEM((R, OUT), jnp.float32),
            pltpu.VMEM((G, L), jnp.int32),
            pltpu.VMEM((G, L), jnp.float32),
            pltpu.SemaphoreType.DMA,
            pltpu.SemaphoreType.DMA,
            pltpu.SemaphoreType.DMA,
            pltpu.SemaphoreType.DMA,
        ],
    )(x_brd, g_idx, g_mask)
    return out.reshape(B, R, G, L)
